# baseline (speedup 1.0000x reference)
import functools

import jax
import jax.numpy as jnp
from jax import lax
from jax.experimental import pallas as pl
from jax.experimental.pallas import tpu as pltpu
from jax.experimental.pallas import tpu_sc as plsc

_VOCAB = 100000
_EMBED = 300
_VBLK = 8192

_NC = 2
_NS = 16
_NW = _NC * _NS
_CH = 128


def _scores_body(p_ref, w_ref, b_ref, out_ref):
    x = p_ref[...]
    w = w_ref[...]
    out_ref[...] = jnp.sum(x * w, axis=0) + b_ref[0]


def _compute_scores(pretrained, W, b):
    nblk = pl.cdiv(_VOCAB, _VBLK)
    return pl.pallas_call(
        _scores_body,
        grid=(nblk,),
        in_specs=[
            pl.BlockSpec((_EMBED, _VBLK), lambda i: (0, i)),
            pl.BlockSpec((_EMBED, 1), lambda i: (0, 0)),
            pl.BlockSpec(memory_space=pltpu.SMEM),
        ],
        out_specs=pl.BlockSpec((_VBLK,), lambda i: (i,)),
        out_shape=jax.ShapeDtypeStruct((_VOCAB,), jnp.float32),
    )(pretrained.T, W.T, b)


def _make_gather(n_total):
    per_w = n_total // _NW
    nch = per_w // _CH
    mesh = plsc.VectorSubcoreMesh(core_axis_name="c", subcore_axis_name="s")

    @functools.partial(
        pl.kernel,
        mesh=mesh,
        out_type=jax.ShapeDtypeStruct((_NW, nch, _CH), jnp.float32),
        scratch_types=[
            pltpu.VMEM((nch, _CH), jnp.int32),
            pltpu.VMEM((nch, _CH), jnp.float32),
            pltpu.SemaphoreType.DMA,
        ],
    )
    def gather(scores_hbm, idx_hbm, out_hbm, idx_v, vals_v, sem):
        wid = lax.axis_index("s") * _NC + lax.axis_index("c")
        pltpu.sync_copy(idx_hbm.at[wid], idx_v)

        def fire(j, carry):
            pltpu.make_async_copy(scores_hbm.at[idx_v.at[j]], vals_v.at[j], sem).start()
            return carry

        lax.fori_loop(0, nch, fire, 0)
        pltpu.make_async_copy(out_hbm.at[wid], vals_v, sem).wait()
        pltpu.sync_copy(vals_v, out_hbm.at[wid])

    return gather


def kernel(input, pretrained, W, b):
    batch, hist = input.shape
    n_total = batch * hist
    scores = _compute_scores(pretrained, W, b)
    idx = input.T.astype(jnp.int32).reshape(_NW, n_total // (_NW * _CH), _CH)
    out = _make_gather(n_total)(scores, idx)
    return out.reshape(hist, batch).T
